# baseline (speedup 1.0000x reference)
import jax
import jax.numpy as jnp
from jax import lax
from jax.experimental import pallas as pl
from jax.experimental.pallas import tpu as pltpu
from jax.experimental.pallas import tpu_sc as plsc

DIM = 64
PADDIM = 128
NW = 32
NBUF = 4
AHEAD = 2


def _emb_body(table_hbm, x_hbm, out_hbm, idx_v, *rest):
    rows = rest[:NBUF]
    gsem = rest[NBUF:2 * NBUF]
    osem = rest[2 * NBUF:3 * NBUF]

    NC, CHUNK = x_hbm.shape
    n_chunks = NC // NW
    wid = lax.axis_index("s") * 2 + lax.axis_index("c")
    base = wid * n_chunks

    pltpu.sync_copy(x_hbm.at[pl.ds(base, n_chunks)], idx_v)

    def fire_gather(g, b):
        return pltpu.async_copy(table_hbm.at[idx_v.at[g]], rows[b], gsem[b])

    def fire_out(g, b):
        dst = out_hbm.at[pl.ds((base + g) * CHUNK, CHUNK)]
        return pltpu.async_copy(rows[b], dst, osem[b])

    for f in range(AHEAD):
        fire_gather(f, f % NBUF)

    def outer(i, carry):
        g0 = i * NBUF
        for b in range(NBUF):
            g = g0 + b
            f = g + AHEAD
            bf = (b + AHEAD) % NBUF

            @pl.when(f < n_chunks)
            def _():
                @pl.when(f >= NBUF)
                def _():
                    pltpu.make_async_copy(
                        rows[bf],
                        out_hbm.at[pl.ds(base * CHUNK, CHUNK)],
                        osem[bf],
                    ).wait()
                fire_gather(f, bf)

            pltpu.make_async_copy(
                table_hbm.at[idx_v.at[g]], rows[b], gsem[b]
            ).wait()
            fire_out(g, b)
        return carry

    lax.fori_loop(0, n_chunks // NBUF, outer, 0)

    for b in range(NBUF):
        pltpu.make_async_copy(
            rows[b],
            out_hbm.at[pl.ds(base * CHUNK, CHUNK)],
            osem[b],
        ).wait()


CHUNK = 128


def kernel(x, table):
    B, L = x.shape
    n_total = B * L
    xi = x.reshape(n_total // CHUNK, CHUNK).astype(jnp.int32)
    tpad = jnp.pad(table.T, ((0, PADDIM - DIM), (0, 0))).T
    scratch = (
        [pltpu.VMEM((n_total // CHUNK // NW, CHUNK), jnp.int32)]
        + [pltpu.VMEM((CHUNK, PADDIM), jnp.float32) for _ in range(NBUF)]
        + [pltpu.SemaphoreType.DMA for _ in range(2 * NBUF)]
    )
    k = pl.kernel(
        _emb_body,
        out_type=jax.ShapeDtypeStruct((n_total, PADDIM), jnp.float32),
        mesh=plsc.VectorSubcoreMesh(core_axis_name="c", subcore_axis_name="s"),
        scratch_types=scratch,
        compiler_params=pltpu.CompilerParams(use_tc_tiling_on_sc=True),
    )
    return k(tpad, xi).reshape(B, L, PADDIM)[..., :DIM]


# speedup vs baseline: 1.0014x; 1.0014x over previous
import jax
import jax.numpy as jnp
from jax import lax
from jax.experimental import pallas as pl
from jax.experimental.pallas import tpu as pltpu
from jax.experimental.pallas import tpu_sc as plsc

DIM = 64
PADDIM = 128
NW = 32
NBUF = 5
AHEAD = 2


def _emb_body(table_hbm, x_hbm, out_hbm, idx_v, *rest):
    rows = rest[:NBUF]
    gsem = rest[NBUF:2 * NBUF]
    osem = rest[2 * NBUF:3 * NBUF]

    NC, CHUNK = x_hbm.shape
    n_chunks = NC // NW
    wid = lax.axis_index("s") * 2 + lax.axis_index("c")
    base = wid * n_chunks

    pltpu.sync_copy(x_hbm.at[pl.ds(base, n_chunks)], idx_v)

    def fire_gather(g, b):
        return pltpu.async_copy(table_hbm.at[idx_v.at[g]], rows[b], gsem[b])

    def fire_out(g, b):
        dst = out_hbm.at[pl.ds((base + g) * CHUNK, CHUNK)]
        return pltpu.async_copy(rows[b], dst, osem[b])

    for f in range(AHEAD):
        fire_gather(f, f % NBUF)

    def outer(i, carry):
        g0 = i * NBUF
        for b in range(NBUF):
            g = g0 + b
            f = g + AHEAD
            bf = (b + AHEAD) % NBUF

            @pl.when(f < n_chunks)
            def _():
                @pl.when(f >= NBUF)
                def _():
                    pltpu.make_async_copy(
                        rows[bf],
                        out_hbm.at[pl.ds(base * CHUNK, CHUNK)],
                        osem[bf],
                    ).wait()
                fire_gather(f, bf)

            pltpu.make_async_copy(
                table_hbm.at[idx_v.at[g]], rows[b], gsem[b]
            ).wait()
            fire_out(g, b)
        return carry

    lax.fori_loop(0, n_chunks // NBUF, outer, 0)

    for b in range(NBUF):
        pltpu.make_async_copy(
            rows[b],
            out_hbm.at[pl.ds(base * CHUNK, CHUNK)],
            osem[b],
        ).wait()


CHUNK = 128


def kernel(x, table):
    B, L = x.shape
    n_total = B * L
    xi = x.reshape(n_total // CHUNK, CHUNK).astype(jnp.int32)
    tpad = jnp.pad(table.T, ((0, PADDIM - DIM), (0, 0))).T
    scratch = (
        [pltpu.VMEM((n_total // CHUNK // NW, CHUNK), jnp.int32)]
        + [pltpu.VMEM((CHUNK, PADDIM), jnp.float32) for _ in range(NBUF)]
        + [pltpu.SemaphoreType.DMA for _ in range(2 * NBUF)]
    )
    k = pl.kernel(
        _emb_body,
        out_type=jax.ShapeDtypeStruct((n_total, PADDIM), jnp.float32),
        mesh=plsc.VectorSubcoreMesh(core_axis_name="c", subcore_axis_name="s"),
        scratch_types=scratch,
        compiler_params=pltpu.CompilerParams(use_tc_tiling_on_sc=True),
    )
    return k(tpad, xi).reshape(B, L, PADDIM)[..., :DIM]


# speedup vs baseline: 1.0025x; 1.0011x over previous
import jax
import jax.numpy as jnp
from jax import lax
from jax.experimental import pallas as pl
from jax.experimental.pallas import tpu as pltpu
from jax.experimental.pallas import tpu_sc as plsc

DIM = 64
PADDIM = 128
NW = 32
NBUF = 4
AHEAD = 2


def _emb_body(table_hbm, x_hbm, out_hbm, idx_v, *rest):
    rows = rest[:NBUF]
    gsem = rest[NBUF:2 * NBUF]
    osem = rest[2 * NBUF:3 * NBUF]

    NC, CHUNK = x_hbm.shape
    n_chunks = NC // NW
    wid = lax.axis_index("s") * 2 + lax.axis_index("c")
    base = wid * n_chunks

    pltpu.sync_copy(x_hbm.at[pl.ds(base, n_chunks)], idx_v)

    def fire_gather(g, b):
        return pltpu.async_copy(table_hbm.at[idx_v.at[g]], rows[b], gsem[b])

    def fire_out(g, b):
        dst = out_hbm.at[pl.ds((base + g) * CHUNK, CHUNK)]
        return pltpu.async_copy(rows[b], dst, osem[b])

    for f in range(AHEAD):
        fire_gather(f, f % NBUF)

    def outer(i, carry):
        g0 = i * NBUF
        for b in range(NBUF):
            g = g0 + b
            f = g + AHEAD
            bf = (b + AHEAD) % NBUF

            @pl.when(f < n_chunks)
            def _():
                @pl.when(f >= NBUF)
                def _():
                    pltpu.make_async_copy(
                        rows[bf],
                        out_hbm.at[pl.ds(base * CHUNK, CHUNK)],
                        osem[bf],
                    ).wait()
                fire_gather(f, bf)

            pltpu.make_async_copy(
                table_hbm.at[idx_v.at[g]], rows[b], gsem[b]
            ).wait()
            fire_out(g, b)
        return carry

    lax.fori_loop(0, n_chunks // NBUF, outer, 0)

    for b in range(NBUF):
        pltpu.make_async_copy(
            rows[b],
            out_hbm.at[pl.ds(base * CHUNK, CHUNK)],
            osem[b],
        ).wait()


CHUNK = 128


def kernel(x, table):
    B, L = x.shape
    n_total = B * L
    xi = x.reshape(n_total // CHUNK, CHUNK).astype(jnp.int32)
    tpad = jnp.pad(table.T, ((0, PADDIM - DIM), (0, 0))).T
    scratch = (
        [pltpu.VMEM((n_total // CHUNK // NW, CHUNK), jnp.int32)]
        + [pltpu.VMEM((CHUNK, PADDIM), jnp.float32) for _ in range(NBUF)]
        + [pltpu.SemaphoreType.DMA for _ in range(2 * NBUF)]
    )
    k = pl.kernel(
        _emb_body,
        out_type=jax.ShapeDtypeStruct((n_total, PADDIM), jnp.float32),
        mesh=plsc.VectorSubcoreMesh(core_axis_name="c", subcore_axis_name="s"),
        scratch_types=scratch,
        compiler_params=pltpu.CompilerParams(use_tc_tiling_on_sc=True),
    )
    return k(tpad, xi).reshape(B, L, PADDIM)[..., :DIM]
